# Initial kernel scaffold; baseline (speedup 1.0000x reference)
import jax
import jax.numpy as jnp
from jax.experimental import pallas as pl


def kernel(x, edge_index, batch, W_emb, b_emb, conv_W, conv_b, W1, b1, W2, b2):
    raise NotImplementedError("write your pallas kernel here")


# speedup vs baseline: 15.9100x; 15.9100x over previous
import functools

import jax
import jax.numpy as jnp
from jax import lax
from jax.experimental import pallas as pl
from jax.experimental.pallas import tpu as pltpu
from jax.experimental.pallas import tpu_sc as plsc

N = 100000
E = 1600000
H = 64
G = 64
NUM_UNIT = 3

SUB = 128
ER = E // SUB
BLK = 16
BLKA = 8
Q = 16
NQ = H // Q

BN = 2000
NB = N // BN

F32 = jnp.float32


def _mesh():
    return plsc.VectorSubcoreMesh(core_axis_name="c", subcore_axis_name="s")


def _deg_body(cols_hbm, zeros_hbm, out_hbm, spm, cstage, onesb, sem_e, sem_s):
    c = lax.axis_index("c")
    s = lax.axis_index("s")
    for i in range(SUB // 16):
        onesb[pl.ds(i * 16, 16)] = jnp.ones((16,), F32)

    @pl.when(s == 0)
    def _():
        pltpu.sync_copy(zeros_hbm, spm)

    plsc.subcore_barrier()

    half = ER // 2
    base_r = c * half
    per = half // 16
    rem = half % 16
    cnt = per + jnp.where(s < rem, 1, 0)
    off = per * s + jnp.minimum(s, rem)
    nfull = cnt // BLK
    tail = cnt - nfull * BLK

    def blk(b, carry):
        r0 = base_r + off + b * BLK
        pltpu.async_copy(cols_hbm.at[pl.ds(r0, BLK)], cstage, sem_e).wait()
        ds_ = [pltpu.async_copy(onesb, spm.at[cstage.at[j, 0]], sem_s, add=True)
               for j in range(BLK)]
        for d in ds_:
            d.wait()
        return carry

    lax.fori_loop(0, nfull, blk, 0)

    def tailf(t, carry):
        r0 = base_r + off + nfull * BLK + t
        pltpu.async_copy(cols_hbm.at[pl.ds(r0, 1)], cstage.at[pl.ds(0, 1)],
                         sem_e).wait()
        pltpu.async_copy(onesb, spm.at[cstage.at[0, 0]], sem_s, add=True).wait()
        return carry

    lax.fori_loop(0, tail, tailf, 0)
    plsc.subcore_barrier()

    @pl.when(s == 0)
    def _():
        pltpu.sync_copy(spm, out_hbm.at[c])


def _sc_degree(cols2d, zeros1):
    return pl.kernel(
        _deg_body,
        out_type=jax.ShapeDtypeStruct((2, N), F32),
        mesh=_mesh(),
        scratch_types=[
            pltpu.VMEM_SHARED((N,), F32),
            pltpu.VMEM((BLK, 1, SUB), jnp.int32),
            pltpu.VMEM((SUB,), F32),
            pltpu.SemaphoreType.DMA,
            pltpu.SemaphoreType.DMA,
        ],
        compiler_params=pltpu.CompilerParams(use_tc_tiling_on_sc=False),
    )(cols2d, zeros1)


def _agg_body(rows_hbm, cols_hbm, g4_hbm, zeros2_hbm, agg_hbm,
              spm, rstage, cstage, gbuf, sem_e, sem_g, sem_s):
    c = lax.axis_index("c")
    s = lax.axis_index("s")
    per = ER // 16
    rem = ER % 16
    cnt = per + jnp.where(s < rem, 1, 0)
    off = per * s + jnp.minimum(s, rem)
    nfull = cnt // BLKA
    tail = cnt - nfull * BLKA

    for ql in range(2):
        q = c * 2 + ql

        @pl.when(s == 0)
        def _():
            pltpu.sync_copy(zeros2_hbm, spm)

        plsc.subcore_barrier()

        def blk(b, carry):
            r0 = off + b * BLKA
            d1 = pltpu.async_copy(rows_hbm.at[pl.ds(r0, BLKA)], rstage, sem_e)
            d2 = pltpu.async_copy(cols_hbm.at[pl.ds(r0, BLKA)], cstage, sem_e)
            d1.wait()
            d2.wait()
            for j in range(BLKA):
                for i in range(SUB // 16):
                    sl = (j, 0, pl.ds(i * 16, 16))
                    rstage[sl] = rstage[sl] * NQ + q
            gds = [pltpu.async_copy(g4_hbm.at[rstage.at[j, 0]], gbuf.at[j], sem_g)
                   for j in range(BLKA)]
            for d in gds:
                d.wait()
            sds = [pltpu.async_copy(gbuf.at[j], spm.at[cstage.at[j, 0]], sem_s,
                                    add=True)
                   for j in range(BLKA)]
            for d in sds:
                d.wait()
            return carry

        lax.fori_loop(0, nfull, blk, 0)

        def tailf(t, carry):
            r0 = off + nfull * BLKA + t
            pltpu.async_copy(rows_hbm.at[pl.ds(r0, 1)],
                             rstage.at[pl.ds(0, 1)], sem_e).wait()
            pltpu.async_copy(cols_hbm.at[pl.ds(r0, 1)],
                             cstage.at[pl.ds(0, 1)], sem_e).wait()
            for i in range(SUB // 16):
                sl = (0, 0, pl.ds(i * 16, 16))
                rstage[sl] = rstage[sl] * NQ + q
            pltpu.async_copy(g4_hbm.at[rstage.at[0, 0]], gbuf.at[0], sem_g).wait()
            pltpu.async_copy(gbuf.at[0], spm.at[cstage.at[0, 0]], sem_s,
                             add=True).wait()
            return carry

        lax.fori_loop(0, tail, tailf, 0)
        plsc.subcore_barrier()

        @pl.when(s == 0)
        def _():
            pltpu.sync_copy(spm, agg_hbm.at[q])

        plsc.subcore_barrier()


def _sc_aggregate(rows2d, cols2d, g, zeros2):
    g4 = g.reshape(N * NQ, Q)
    agg4 = pl.kernel(
        _agg_body,
        out_type=jax.ShapeDtypeStruct((NQ, N, Q), F32),
        mesh=_mesh(),
        scratch_types=[
            pltpu.VMEM_SHARED((N, Q), F32),
            pltpu.VMEM((BLKA, 1, SUB), jnp.int32),
            pltpu.VMEM((BLKA, 1, SUB), jnp.int32),
            pltpu.VMEM((BLKA, SUB, Q), F32),
            pltpu.SemaphoreType.DMA,
            pltpu.SemaphoreType.DMA,
            pltpu.SemaphoreType.DMA,
        ],
        compiler_params=pltpu.CompilerParams(use_tc_tiling_on_sc=False),
    )(rows2d, cols2d, g4, zeros2)
    return agg4.transpose(1, 0, 2).reshape(N, H)


def _tc0_body(x_ref, wemb_ref, bemb_ref, w0_ref, degt_ref, g1_ref, dis_ref):
    dt = degt_ref[...]
    deg = dt[:, 0:1] + dt[:, 1:2] + 1.0
    dis = lax.rsqrt(deg)
    h0 = jnp.dot(x_ref[...], wemb_ref[...],
                 preferred_element_type=F32) + bemb_ref[...]
    g1_ref[...] = jnp.dot(h0, w0_ref[...], preferred_element_type=F32) * dis
    dis_ref[...] = dis


def _tc0(x, W_emb, b_emb2, W0, deg_t):
    return pl.pallas_call(
        _tc0_body,
        grid=(NB,),
        in_specs=[
            pl.BlockSpec((BN, 4), lambda i: (i, 0)),
            pl.BlockSpec((4, H), lambda i: (0, 0)),
            pl.BlockSpec((1, H), lambda i: (0, 0)),
            pl.BlockSpec((H, H), lambda i: (0, 0)),
            pl.BlockSpec((BN, 2), lambda i: (i, 0)),
        ],
        out_specs=[
            pl.BlockSpec((BN, H), lambda i: (i, 0)),
            pl.BlockSpec((BN, 1), lambda i: (i, 0)),
        ],
        out_shape=[
            jax.ShapeDtypeStruct((N, H), F32),
            jax.ShapeDtypeStruct((N, 1), F32),
        ],
    )(x, W_emb, b_emb2, W0, deg_t)


def _tcl_body(agg_ref, g_ref, dis_ref, b_ref, w_ref, gout_ref):
    dis = dis_ref[...]
    h = jnp.maximum(dis * (agg_ref[...] + g_ref[...]) + b_ref[...], 0.0)
    gout_ref[...] = jnp.dot(h, w_ref[...], preferred_element_type=F32) * dis


def _tc_layer(agg, g, dis, b2, W_next):
    return pl.pallas_call(
        _tcl_body,
        grid=(NB,),
        in_specs=[
            pl.BlockSpec((BN, H), lambda i: (i, 0)),
            pl.BlockSpec((BN, H), lambda i: (i, 0)),
            pl.BlockSpec((BN, 1), lambda i: (i, 0)),
            pl.BlockSpec((1, H), lambda i: (0, 0)),
            pl.BlockSpec((H, H), lambda i: (0, 0)),
        ],
        out_specs=pl.BlockSpec((BN, H), lambda i: (i, 0)),
        out_shape=jax.ShapeDtypeStruct((N, H), F32),
    )(agg, g, dis, b2, W_next)


def _tc3_body(agg_ref, g_ref, dis_ref, b_ref, batch_ref,
              w1_ref, b1_ref, w2_ref, b2_ref, out_ref, sums_s, cnt_s):
    i = pl.program_id(0)

    @pl.when(i == 0)
    def _():
        sums_s[...] = jnp.zeros((G, H), F32)
        cnt_s[...] = jnp.zeros((G, 1), F32)

    h = jnp.maximum(dis_ref[...] * (agg_ref[...] + g_ref[...]) + b_ref[...],
                    0.0)
    iota = lax.broadcasted_iota(jnp.int32, (1, G), 1)
    onehot = (batch_ref[...] == iota).astype(F32)
    dn = (((0,), (0,)), ((), ()))
    sums_s[...] += lax.dot_general(onehot, h, dimension_numbers=dn,
                                   preferred_element_type=F32)
    cnt_s[...] += lax.dot_general(onehot, jnp.ones((BN, 1), F32),
                                  dimension_numbers=dn,
                                  preferred_element_type=F32)

    @pl.when(i == NB - 1)
    def _():
        gx = sums_s[...] / jnp.maximum(cnt_s[...], 1.0)
        z = jnp.maximum(jnp.dot(gx, w1_ref[...],
                                preferred_element_type=F32) + b1_ref[...], 0.0)
        out_ref[...] = jnp.dot(z, w2_ref[...],
                               preferred_element_type=F32) + b2_ref[...]


def _tc3(agg, g, dis, b2d, batch2, W1, b1_2, W2, b2_2):
    return pl.pallas_call(
        _tc3_body,
        grid=(NB,),
        in_specs=[
            pl.BlockSpec((BN, H), lambda i: (i, 0)),
            pl.BlockSpec((BN, H), lambda i: (i, 0)),
            pl.BlockSpec((BN, 1), lambda i: (i, 0)),
            pl.BlockSpec((1, H), lambda i: (0, 0)),
            pl.BlockSpec((BN, 1), lambda i: (i, 0)),
            pl.BlockSpec((H, H), lambda i: (0, 0)),
            pl.BlockSpec((1, H), lambda i: (0, 0)),
            pl.BlockSpec((H, 3), lambda i: (0, 0)),
            pl.BlockSpec((1, 3), lambda i: (0, 0)),
        ],
        out_specs=pl.BlockSpec((G, 3), lambda i: (0, 0)),
        out_shape=jax.ShapeDtypeStruct((G, 3), F32),
        scratch_shapes=[
            pltpu.VMEM((G, H), F32),
            pltpu.VMEM((G, 1), F32),
        ],
    )(agg, g, dis, b2d, batch2, W1, b1_2, W2, b2_2)


def kernel(x, edge_index, batch, W_emb, b_emb, conv_W, conv_b, W1, b1, W2, b2):
    rows2d = edge_index[0].reshape(ER, 1, SUB)
    cols2d = edge_index[1].reshape(ER, 1, SUB)
    zeros1 = jnp.zeros((N,), F32)
    zeros2 = jnp.zeros((N, Q), F32)

    deg_p = _sc_degree(cols2d, zeros1)
    deg_t = deg_p.T

    g, dis = _tc0(x, W_emb, b_emb.reshape(1, H), conv_W[0], deg_t)
    for i in range(NUM_UNIT):
        agg = _sc_aggregate(rows2d, cols2d, g, zeros2)
        b2d = conv_b[i].reshape(1, H)
        if i < NUM_UNIT - 1:
            g = _tc_layer(agg, g, dis, b2d, conv_W[i + 1])
        else:
            pred = _tc3(agg, g, dis, b2d, batch.reshape(N, 1),
                        W1, b1.reshape(1, H), W2, b2.reshape(1, 3))
    return pred
